# baseline (speedup 1.0000x reference)
import functools

import jax
import jax.numpy as jnp
from jax import lax
from jax.experimental import pallas as pl
from jax.experimental.pallas import tpu as pltpu
from jax.experimental.pallas import tpu_sc as plsc

L = 16
NW = 32
B, N, H, F = 4, 4096, 100, 6
LANES = 128
CH_H = 20
NCH = H // CH_H
MAPW = 256 * 256
VC = 1.5 / (10.0 ** 0.5)


def _sqrt16(a, iters=2):
    i = plsc.bitcast(a, jnp.int32)
    i = 0x5F3759DF - lax.shift_right_logical(i, 1)
    y = plsc.bitcast(i, jnp.float32)
    half = 0.5 * a
    for _ in range(iters):
        y = y * (1.5 - half * y * y)
    return a * y


_mesh = plsc.VectorSubcoreMesh(core_axis_name="c", subcore_axis_name="s")

_CHUNK = pltpu.VMEM((CH_H, B, LANES), jnp.float32)


@functools.partial(
    pl.kernel,
    out_type=jax.ShapeDtypeStruct((B, N), jnp.float32),
    mesh=_mesh,
    scratch_types=[
        pltpu.VMEM((MAPW,), jnp.float32),
        _CHUNK, _CHUNK,
        _CHUNK, _CHUNK,
        pltpu.VMEM((B, LANES), jnp.float32),
        pltpu.VMEM((2 * L,), jnp.float32),
        pltpu.SemaphoreType.DMA,
        pltpu.SemaphoreType.DMA,
        pltpu.SemaphoreType.DMA,
    ],
    compiler_params=pltpu.CompilerParams(needs_layout_passes=False),
)
def _path_kernel(states_hbm, bev_hbm, goal_hbm, out_hbm,
                 bev_v, x0, y0, x1, y1, out_v, goal_v,
                 sem_bev, sem0, sem1):
    wid = lax.axis_index("s") * 2 + lax.axis_index("c")
    col = wid * LANES
    bufs = ((x0, y0), (x1, y1))
    sems = (sem0, sem1)

    def start(c):
        p = c % 2
        return [
            pltpu.async_copy(
                states_hbm.at[pl.ds(c * CH_H, CH_H), f, :, pl.ds(col, LANES)],
                bufs[p][f], sems[p])
            for f in (0, 1)
        ]

    bev_cp = pltpu.async_copy(bev_hbm, bev_v, sem_bev)
    cps = [None] * NCH
    cps[0] = start(0)
    pltpu.sync_copy(goal_hbm, goal_v)
    gx = goal_v[pl.ds(0, L)]
    gy = goal_v[pl.ds(L, L)]
    bev_cp.wait()

    for c in range(NCH):
        for cp in cps[c]:
            cp.wait()
        if c + 1 < NCH:
            cps[c + 1] = start(c + 1)
        xb, yb = bufs[c % 2]
        first = c == 0

        def group_body(g, carry, xb=xb, yb=yb, first=first):
            b = lax.shift_right_logical(g, 3)
            l0 = lax.shift_left(g & 7, 4)

            def step_body(h, acc):
                xv = xb[h, b, pl.ds(l0, L)]
                yv = yb[h, b, pl.ds(l0, L)]
                fx = jnp.minimum(jnp.maximum(xv + 128.0, 0.0), 255.0)
                fy = jnp.minimum(jnp.maximum(yv + 128.0, 0.0), 255.0)
                flat = fy.astype(jnp.int32) * 256 + fx.astype(jnp.int32)
                return acc + plsc.load_gather(bev_v, [flat])

            acc = lax.fori_loop(0, CH_H, step_body,
                                jnp.zeros((L,), jnp.float32), unroll=CH_H)
            if first:
                out_v[b, pl.ds(l0, L)] = acc * (1.0 / 255.0)
            else:
                out_v[b, pl.ds(l0, L)] += acc * (1.0 / 255.0)
            return carry

        lax.fori_loop(0, NW, group_body, 0)

    xl, yl = bufs[(NCH - 1) % 2]

    def term_body(g, carry):
        b = lax.shift_right_logical(g, 3)
        l0 = lax.shift_left(g & 7, 4)
        dx = xl[CH_H - 1, b, pl.ds(l0, L)] - gx
        dy = yl[CH_H - 1, b, pl.ds(l0, L)] - gy
        out_v[b, pl.ds(l0, L)] += _sqrt16(dx * dx + dy * dy)
        return carry

    lax.fori_loop(0, NW, term_body, 0)
    pltpu.sync_copy(out_v, out_hbm.at[:, pl.ds(col, LANES)])


def _vel_body(v_ref, out_ref):
    h = pl.program_id(0)

    @pl.when(h == 0)
    def _():
        out_ref[...] = jnp.zeros_like(out_ref)

    out_ref[...] += VC * jnp.sqrt(jnp.abs(10.0 - v_ref[0]))


_vel_kernel = pl.pallas_call(
    _vel_body,
    grid=(H,),
    in_specs=[pl.BlockSpec((1, B, N), lambda h: (6 * h + 3, 0, 0))],
    out_specs=pl.BlockSpec((B, N), lambda h: (0, 0)),
    out_shape=jax.ShapeDtypeStruct((B, N), jnp.float32),
)


def kernel(states, controls, bev_path, goal_state):
    del controls
    states_t = jnp.transpose(states, (2, 3, 0, 1)).reshape(H, F, B, N)
    states_r = states_t.reshape(H * F, B, N)
    bev_flat = bev_path.reshape(-1)
    goal2 = jnp.concatenate([
        jnp.full((L,), goal_state[0], jnp.float32),
        jnp.full((L,), goal_state[1], jnp.float32),
    ])
    path_goal = _path_kernel(states_t, bev_flat, goal2)
    vel = _vel_kernel(states_r)
    return path_goal + vel


# speedup vs baseline: 1.3994x; 1.3994x over previous
import functools

import jax
import jax.numpy as jnp
from jax import lax
from jax.experimental import pallas as pl
from jax.experimental.pallas import tpu as pltpu
from jax.experimental.pallas import tpu_sc as plsc

L = 16
NW = 32
B, N, H, F = 4, 4096, 100, 6
LANES = 128
CH_H = 20
NCH = H // CH_H
MAPW = 256 * 256
VC = 1.5 / (10.0 ** 0.5)


def _sqrt16(a, iters=2):
    i = plsc.bitcast(a, jnp.int32)
    i = 0x5F3759DF - lax.shift_right_logical(i, 1)
    y = plsc.bitcast(i, jnp.float32)
    half = 0.5 * a
    for _ in range(iters):
        y = y * (1.5 - half * y * y)
    return a * y


_mesh = plsc.VectorSubcoreMesh(core_axis_name="c", subcore_axis_name="s")

_CHUNK = pltpu.VMEM((CH_H, B, LANES), jnp.float32)


@functools.partial(
    pl.kernel,
    out_type=jax.ShapeDtypeStruct((B, N), jnp.float32),
    mesh=_mesh,
    scratch_types=[
        pltpu.VMEM((MAPW,), jnp.float32),
        _CHUNK, _CHUNK,
        _CHUNK, _CHUNK,
        pltpu.VMEM((B, LANES), jnp.float32),
        pltpu.VMEM((2 * L,), jnp.float32),
        pltpu.SemaphoreType.DMA,
        pltpu.SemaphoreType.DMA,
        pltpu.SemaphoreType.DMA,
    ],
    compiler_params=pltpu.CompilerParams(needs_layout_passes=False),
)
def _path_kernel(states_hbm, bev_hbm, goal_hbm, out_hbm,
                 bev_v, x0, y0, x1, y1, out_v, goal_v,
                 sem_bev, sem0, sem1):
    wid = lax.axis_index("s") * 2 + lax.axis_index("c")
    col = wid * LANES
    bufs = ((x0, y0), (x1, y1))
    sems = (sem0, sem1)

    def start(c):
        p = c % 2
        return [
            pltpu.async_copy(
                states_hbm.at[pl.ds(c * CH_H, CH_H), f, :, pl.ds(col, LANES)],
                bufs[p][f], sems[p])
            for f in (0, 1)
        ]

    bev_cp = pltpu.async_copy(bev_hbm, bev_v, sem_bev)
    cps = [None] * NCH
    cps[0] = start(0)
    pltpu.sync_copy(goal_hbm, goal_v)
    gx = goal_v[pl.ds(0, L)]
    gy = goal_v[pl.ds(L, L)]
    bev_cp.wait()

    for c in range(NCH):
        for cp in cps[c]:
            cp.wait()
        if c + 1 < NCH:
            cps[c + 1] = start(c + 1)
        xb, yb = bufs[c % 2]
        first = c == 0

        def group_body(g, carry, xb=xb, yb=yb, first=first):
            b = lax.shift_right_logical(g, 3)
            l0 = lax.shift_left(g & 7, 4)

            def step_body(h, acc):
                xv = xb[h, b, pl.ds(l0, L)]
                yv = yb[h, b, pl.ds(l0, L)]
                fx = jnp.minimum(jnp.maximum(xv + 128.0, 0.0), 255.0)
                fy = jnp.minimum(jnp.maximum(yv + 128.0, 0.0), 255.0)
                flat = fy.astype(jnp.int32) * 256 + fx.astype(jnp.int32)
                return acc + plsc.load_gather(bev_v, [flat])

            acc = lax.fori_loop(0, CH_H, step_body,
                                jnp.zeros((L,), jnp.float32), unroll=CH_H)
            if first:
                out_v[b, pl.ds(l0, L)] = acc * (1.0 / 255.0)
            else:
                out_v[b, pl.ds(l0, L)] += acc * (1.0 / 255.0)
            return carry

        lax.fori_loop(0, NW, group_body, 0)

    xl, yl = bufs[(NCH - 1) % 2]

    def term_body(g, carry):
        b = lax.shift_right_logical(g, 3)
        l0 = lax.shift_left(g & 7, 4)
        dx = xl[CH_H - 1, b, pl.ds(l0, L)] - gx
        dy = yl[CH_H - 1, b, pl.ds(l0, L)] - gy
        out_v[b, pl.ds(l0, L)] += _sqrt16(dx * dx + dy * dy)
        return carry

    lax.fori_loop(0, NW, term_body, 0)
    pltpu.sync_copy(out_v, out_hbm.at[:, pl.ds(col, LANES)])


def _vel_body(v_ref, out_ref):
    out_ref[...] = VC * jnp.sum(jnp.sqrt(jnp.abs(10.0 - v_ref[:, 0])), axis=0)


_vel_kernel = pl.pallas_call(
    _vel_body,
    grid=(1,),
    in_specs=[pl.BlockSpec((H, 1, B, N), lambda i: (0, 3, 0, 0))],
    out_specs=pl.BlockSpec((B, N), lambda i: (0, 0)),
    out_shape=jax.ShapeDtypeStruct((B, N), jnp.float32),
)


def kernel(states, controls, bev_path, goal_state):
    del controls
    states_t = jnp.transpose(states, (2, 3, 0, 1)).reshape(H, F, B, N)
    bev_flat = bev_path.reshape(-1)
    goal2 = jnp.concatenate([
        jnp.full((L,), goal_state[0], jnp.float32),
        jnp.full((L,), goal_state[1], jnp.float32),
    ])
    path_goal = _path_kernel(states_t, bev_flat, goal2)
    vel = _vel_kernel(states_t)
    return path_goal + vel
